# Initial kernel scaffold; baseline (speedup 1.0000x reference)
import jax
import jax.numpy as jnp
from jax.experimental import pallas as pl


def kernel(x_embed, prompt_pool, prompt_key):
    raise NotImplementedError("write your pallas kernel here")


# speedup vs baseline: 1.4606x; 1.4606x over previous
import functools

import jax
import jax.numpy as jnp
from jax import lax
from jax.experimental import pallas as pl
from jax.experimental.pallas import tpu as pltpu
from jax.experimental.pallas import tpu_sc as plsc

B = 16
SEQ = 2048
D = 1024
POOL = 64
LEN = 11
K = 8
PROWS = K * LEN
BS = 128
NSTEP = SEQ // BS


def _stream_body(x_ref, pk_ref, out_ref, sim_ref, pn_ref, xsum_ref,
                 mean_ref, sem):
    s = pl.program_id(0)
    cp = pltpu.make_async_copy(
        x_ref, out_ref.at[:, pl.ds(PROWS + s * BS, BS), :], sem)
    cp.start()
    m = jnp.max(x_ref[...], axis=1)

    @pl.when(s == 0)
    def _():
        mean_ref[...] = m

    @pl.when(s > 0)
    def _():
        mean_ref[...] = jnp.maximum(mean_ref[...], m)

    @pl.when(s == NSTEP - 1)
    def _():
        mean = mean_ref[...]
        xn = mean * lax.rsqrt(
            jnp.maximum(jnp.sum(mean * mean, axis=1, keepdims=True), 1e-12))
        pk = pk_ref[...]
        pn = pk * lax.rsqrt(
            jnp.maximum(jnp.sum(pk * pk, axis=1, keepdims=True), 1e-12))
        pn_ref[...] = pn
        sim_ref[...] = lax.dot_general(
            xn, pn, (((1,), (1,)), ((), ())),
            preferred_element_type=jnp.float32)
        xsum_ref[...] = jnp.sum(xn, axis=0, keepdims=True)

    cp.wait()


_stream_call = pl.pallas_call(
    _stream_body,
    grid=(NSTEP,),
    in_specs=[
        pl.BlockSpec((B, BS, D), lambda s: (0, s, 0)),
        pl.BlockSpec((POOL, D), lambda s: (0, 0)),
    ],
    out_specs=[
        pl.BlockSpec(memory_space=pl.ANY),
        pl.BlockSpec((B, POOL), lambda s: (0, 0)),
        pl.BlockSpec((POOL, D), lambda s: (0, 0)),
        pl.BlockSpec((1, D), lambda s: (0, 0)),
    ],
    out_shape=[
        jax.ShapeDtypeStruct((B, PROWS + SEQ, D), jnp.float32),
        jax.ShapeDtypeStruct((B, POOL), jnp.float32),
        jax.ShapeDtypeStruct((POOL, D), jnp.float32),
        jax.ShapeDtypeStruct((1, D), jnp.float32),
    ],
    scratch_shapes=[
        pltpu.VMEM((B, D), jnp.float32),
        pltpu.SemaphoreType.DMA,
    ],
)


def _sc_body(sim_hbm, pn_hbm, xsum_hbm, pool_hbm,
             prompts_hbm, idx_hbm, bkn_hbm, rs_hbm,
             sim_v, idx96_v, prompts_v, knorm_v, maj16_v, idxout_v,
             rs_v, xsum_v, sem):
    wid = lax.axis_index("s") * 2 + lax.axis_index("c")
    iota = lax.iota(jnp.int32, 16)
    neg = jnp.float32(-1e30)

    pltpu.sync_copy(sim_hbm, sim_v)

    def row_body(r, carry):
        c = list(carry)
        v = [sim_v[pl.ds(r * POOL + 16 * j, 16)] for j in range(4)]
        for _ in range(K):
            m = jnp.max(jnp.maximum(jnp.maximum(v[0], v[1]),
                                    jnp.maximum(v[2], v[3])))
            cand = [jnp.where(v[j] == m, iota + 16 * j, POOL)
                    for j in range(4)]
            idxk = jnp.min(jnp.minimum(jnp.minimum(cand[0], cand[1]),
                                       jnp.minimum(cand[2], cand[3])))
            for j in range(4):
                hit = (iota + 16 * j) == idxk
                c[j] = jnp.where(hit, c[j] + 1, c[j])
                v[j] = jnp.where(hit, neg, v[j])
        return tuple(c)

    zero = jnp.zeros((16,), jnp.int32)
    counts = lax.fori_loop(0, B, row_body, (zero, zero, zero, zero))

    keys = [counts[j] * POOL + (POOL - 1 - (iota + 16 * j))
            for j in range(4)]
    majors = []
    for _ in range(K):
        m = jnp.max(jnp.maximum(jnp.maximum(keys[0], keys[1]),
                                jnp.maximum(keys[2], keys[3])))
        majors.append(POOL - 1 - jnp.bitwise_and(m, POOL - 1))
        keys = [jnp.where(keys[j] == m, -1, keys[j]) for j in range(4)]

    @pl.when(wid == 0)
    def _():
        for vb in range(6):
            e = iota + 16 * vb
            kq = jnp.minimum(e // LEN, K - 1)
            l = e - kq * LEN
            idv = jnp.zeros((16,), jnp.int32)
            for k in range(K):
                idv = jnp.where(kq == k, majors[k], idv)
            idx96_v[pl.ds(16 * vb, 16)] = jnp.minimum(
                idv * LEN + l, POOL * LEN - 1)
        pltpu.async_copy(pool_hbm.at[idx96_v], prompts_v, sem).wait()
        pltpu.sync_copy(prompts_v.at[pl.ds(0, PROWS)], prompts_hbm)

    @pl.when(wid == 2)
    def _():
        pat = jnp.zeros((16,), jnp.int32)
        for k in range(K):
            pat = jnp.where(jnp.bitwise_and(iota, K - 1) == k,
                            majors[k], pat)
        for vb in range(8):
            idxout_v[pl.ds(16 * vb, 16)] = pat
        pltpu.sync_copy(idxout_v, idx_hbm)

    @pl.when(wid >= 16)
    def _():
        mv = jnp.zeros((16,), jnp.int32)
        for k in range(K):
            mv = jnp.where(iota == k, majors[k], mv)
        mv = jnp.where(iota >= K, majors[K - 1], mv)
        maj16_v[...] = mv
        pltpu.async_copy(pn_hbm.at[maj16_v], knorm_v, sem).wait()
        pltpu.sync_copy(knorm_v.at[pl.ds(0, K)], bkn_hbm.at[wid - 16])

    @pl.when(wid == 16)
    def _():
        pltpu.sync_copy(xsum_hbm, xsum_v)

        def dot_body(d, acc):
            xa = xsum_v[pl.ds(16 * d, 16)]
            ks = knorm_v[0, pl.ds(16 * d, 16)]
            for k in range(1, K):
                ks = ks + knorm_v[k, pl.ds(16 * d, 16)]
            return acc + xa * ks

        acc = lax.fori_loop(0, D // 16, dot_body,
                            jnp.zeros((16,), jnp.float32))
        rsum = jnp.sum(acc) * jnp.float32(1.0 / B)
        rs_v[...] = jnp.where(iota == 0, rsum, 0.0)
        pltpu.sync_copy(rs_v, rs_hbm)


@functools.cache
def _get_sc_call():
    return functools.partial(
        pl.kernel,
        out_type=[
            jax.ShapeDtypeStruct((PROWS, D), jnp.float32),
            jax.ShapeDtypeStruct((B * K,), jnp.int32),
            jax.ShapeDtypeStruct((B, K, D), jnp.float32),
            jax.ShapeDtypeStruct((16,), jnp.float32),
        ],
        mesh=plsc.VectorSubcoreMesh(
            core_axis_name="c", subcore_axis_name="s"),
        compiler_params=pltpu.CompilerParams(needs_layout_passes=False),
        scratch_types=[
            pltpu.VMEM((B * POOL,), jnp.float32),
            pltpu.VMEM((96,), jnp.int32),
            pltpu.VMEM((96, D), jnp.float32),
            pltpu.VMEM((16, D), jnp.float32),
            pltpu.VMEM((16,), jnp.int32),
            pltpu.VMEM((B * K,), jnp.int32),
            pltpu.VMEM((16,), jnp.float32),
            pltpu.VMEM((D,), jnp.float32),
            pltpu.SemaphoreType.DMA,
        ],
    )(_sc_body)


def _insert_body(prompts_ref, outin_ref, out_ref, sem):
    del outin_ref
    cps = [pltpu.make_async_copy(
        prompts_ref, out_ref.at[b, pl.ds(0, PROWS), :], sem)
        for b in range(B)]
    for cp in cps:
        cp.start()
    for cp in cps:
        cp.wait()


_insert_call = pl.pallas_call(
    _insert_body,
    in_specs=[
        pl.BlockSpec((PROWS, D), lambda: (0, 0)),
        pl.BlockSpec(memory_space=pl.ANY),
    ],
    out_specs=pl.BlockSpec(memory_space=pl.ANY),
    out_shape=jax.ShapeDtypeStruct((B, PROWS + SEQ, D), jnp.float32),
    scratch_shapes=[pltpu.SemaphoreType.DMA],
    input_output_aliases={1: 0},
)


def kernel(x_embed, prompt_pool, prompt_key):
    out0, sim, pnorm, xsum = _stream_call(x_embed, prompt_key)
    prompts, idx128, bkn, rs = _get_sc_call()(
        sim.reshape(B * POOL), pnorm, xsum.reshape(D),
        prompt_pool.reshape(POOL * LEN, D))
    out = _insert_call(prompts, out0)
    return (out, rs[0], sim, idx128.reshape(B, K), bkn)


# speedup vs baseline: 1.4686x; 1.0054x over previous
import functools

import jax
import jax.numpy as jnp
from jax import lax
from jax.experimental import pallas as pl
from jax.experimental.pallas import tpu as pltpu
from jax.experimental.pallas import tpu_sc as plsc

B = 16
SEQ = 2048
D = 1024
POOL = 64
LEN = 11
K = 8
PROWS = K * LEN
BS = 256
NSTEP = SEQ // BS


def _stream_body(x_ref, pk_ref, out_ref, sim_ref, pn_ref, xsum_ref,
                 mean_ref, sem):
    s = pl.program_id(0)
    cp = pltpu.make_async_copy(
        x_ref, out_ref.at[:, pl.ds(PROWS + s * BS, BS), :], sem)
    cp.start()
    m = jnp.max(x_ref[...], axis=1)

    @pl.when(s == 0)
    def _():
        mean_ref[...] = m

    @pl.when(s > 0)
    def _():
        mean_ref[...] = jnp.maximum(mean_ref[...], m)

    @pl.when(s == NSTEP - 1)
    def _():
        mean = mean_ref[...]
        xn = mean * lax.rsqrt(
            jnp.maximum(jnp.sum(mean * mean, axis=1, keepdims=True), 1e-12))
        pk = pk_ref[...]
        pn = pk * lax.rsqrt(
            jnp.maximum(jnp.sum(pk * pk, axis=1, keepdims=True), 1e-12))
        pn_ref[...] = pn
        sim_ref[...] = lax.dot_general(
            xn, pn, (((1,), (1,)), ((), ())),
            preferred_element_type=jnp.float32)
        xsum_ref[...] = jnp.sum(xn, axis=0, keepdims=True)

    cp.wait()


_stream_call = pl.pallas_call(
    _stream_body,
    grid=(NSTEP,),
    in_specs=[
        pl.BlockSpec((B, BS, D), lambda s: (0, s, 0)),
        pl.BlockSpec((POOL, D), lambda s: (0, 0)),
    ],
    out_specs=[
        pl.BlockSpec(memory_space=pl.ANY),
        pl.BlockSpec((B, POOL), lambda s: (0, 0)),
        pl.BlockSpec((POOL, D), lambda s: (0, 0)),
        pl.BlockSpec((1, D), lambda s: (0, 0)),
    ],
    out_shape=[
        jax.ShapeDtypeStruct((B, PROWS + SEQ, D), jnp.float32),
        jax.ShapeDtypeStruct((B, POOL), jnp.float32),
        jax.ShapeDtypeStruct((POOL, D), jnp.float32),
        jax.ShapeDtypeStruct((1, D), jnp.float32),
    ],
    scratch_shapes=[
        pltpu.VMEM((B, D), jnp.float32),
        pltpu.SemaphoreType.DMA,
    ],
)


def _sc_body(sim_hbm, pn_hbm, xsum_hbm, pool_hbm,
             prompts_hbm, idx_hbm, bkn_hbm, rs_hbm,
             sim_v, idx96_v, prompts_v, knorm_v, maj16_v, idxout_v,
             rs_v, xsum_v, sem):
    wid = lax.axis_index("s") * 2 + lax.axis_index("c")
    iota = lax.iota(jnp.int32, 16)
    neg = jnp.float32(-1e30)

    pltpu.sync_copy(sim_hbm, sim_v)

    def row_body(r, carry):
        c = list(carry)
        v = [sim_v[pl.ds(r * POOL + 16 * j, 16)] for j in range(4)]
        for _ in range(K):
            m = jnp.max(jnp.maximum(jnp.maximum(v[0], v[1]),
                                    jnp.maximum(v[2], v[3])))
            cand = [jnp.where(v[j] == m, iota + 16 * j, POOL)
                    for j in range(4)]
            idxk = jnp.min(jnp.minimum(jnp.minimum(cand[0], cand[1]),
                                       jnp.minimum(cand[2], cand[3])))
            for j in range(4):
                hit = (iota + 16 * j) == idxk
                c[j] = jnp.where(hit, c[j] + 1, c[j])
                v[j] = jnp.where(hit, neg, v[j])
        return tuple(c)

    zero = jnp.zeros((16,), jnp.int32)
    counts = lax.fori_loop(0, B, row_body, (zero, zero, zero, zero))

    keys = [counts[j] * POOL + (POOL - 1 - (iota + 16 * j))
            for j in range(4)]
    majors = []
    for _ in range(K):
        m = jnp.max(jnp.maximum(jnp.maximum(keys[0], keys[1]),
                                jnp.maximum(keys[2], keys[3])))
        majors.append(POOL - 1 - jnp.bitwise_and(m, POOL - 1))
        keys = [jnp.where(keys[j] == m, -1, keys[j]) for j in range(4)]

    @pl.when(wid == 0)
    def _():
        for vb in range(6):
            e = iota + 16 * vb
            kq = jnp.minimum(e // LEN, K - 1)
            l = e - kq * LEN
            idv = jnp.zeros((16,), jnp.int32)
            for k in range(K):
                idv = jnp.where(kq == k, majors[k], idv)
            idx96_v[pl.ds(16 * vb, 16)] = jnp.minimum(
                idv * LEN + l, POOL * LEN - 1)
        pltpu.async_copy(pool_hbm.at[idx96_v], prompts_v, sem).wait()
        pltpu.sync_copy(prompts_v.at[pl.ds(0, PROWS)], prompts_hbm)

    @pl.when(wid == 2)
    def _():
        pat = jnp.zeros((16,), jnp.int32)
        for k in range(K):
            pat = jnp.where(jnp.bitwise_and(iota, K - 1) == k,
                            majors[k], pat)
        for vb in range(8):
            idxout_v[pl.ds(16 * vb, 16)] = pat
        pltpu.sync_copy(idxout_v, idx_hbm)

    @pl.when(wid >= 16)
    def _():
        mv = jnp.zeros((16,), jnp.int32)
        for k in range(K):
            mv = jnp.where(iota == k, majors[k], mv)
        mv = jnp.where(iota >= K, majors[K - 1], mv)
        maj16_v[...] = mv
        pltpu.async_copy(pn_hbm.at[maj16_v], knorm_v, sem).wait()
        pltpu.sync_copy(knorm_v.at[pl.ds(0, K)], bkn_hbm.at[wid - 16])

    @pl.when(wid == 16)
    def _():
        pltpu.sync_copy(xsum_hbm, xsum_v)

        def dot_body(d, acc):
            xa = xsum_v[pl.ds(16 * d, 16)]
            ks = knorm_v[0, pl.ds(16 * d, 16)]
            for k in range(1, K):
                ks = ks + knorm_v[k, pl.ds(16 * d, 16)]
            return acc + xa * ks

        acc = lax.fori_loop(0, D // 16, dot_body,
                            jnp.zeros((16,), jnp.float32))
        rsum = jnp.sum(acc) * jnp.float32(1.0 / B)
        rs_v[...] = jnp.where(iota == 0, rsum, 0.0)
        pltpu.sync_copy(rs_v, rs_hbm)


@functools.cache
def _get_sc_call():
    return functools.partial(
        pl.kernel,
        out_type=[
            jax.ShapeDtypeStruct((PROWS, D), jnp.float32),
            jax.ShapeDtypeStruct((B * K,), jnp.int32),
            jax.ShapeDtypeStruct((B, K, D), jnp.float32),
            jax.ShapeDtypeStruct((16,), jnp.float32),
        ],
        mesh=plsc.VectorSubcoreMesh(
            core_axis_name="c", subcore_axis_name="s"),
        compiler_params=pltpu.CompilerParams(needs_layout_passes=False),
        scratch_types=[
            pltpu.VMEM((B * POOL,), jnp.float32),
            pltpu.VMEM((96,), jnp.int32),
            pltpu.VMEM((96, D), jnp.float32),
            pltpu.VMEM((16, D), jnp.float32),
            pltpu.VMEM((16,), jnp.int32),
            pltpu.VMEM((B * K,), jnp.int32),
            pltpu.VMEM((16,), jnp.float32),
            pltpu.VMEM((D,), jnp.float32),
            pltpu.SemaphoreType.DMA,
        ],
    )(_sc_body)


def _insert_body(prompts_ref, outin_ref, out_ref, sem):
    del outin_ref
    cps = [pltpu.make_async_copy(
        prompts_ref, out_ref.at[b, pl.ds(0, PROWS), :], sem)
        for b in range(B)]
    for cp in cps:
        cp.start()
    for cp in cps:
        cp.wait()


_insert_call = pl.pallas_call(
    _insert_body,
    in_specs=[
        pl.BlockSpec((PROWS, D), lambda: (0, 0)),
        pl.BlockSpec(memory_space=pl.ANY),
    ],
    out_specs=pl.BlockSpec(memory_space=pl.ANY),
    out_shape=jax.ShapeDtypeStruct((B, PROWS + SEQ, D), jnp.float32),
    scratch_shapes=[pltpu.SemaphoreType.DMA],
    input_output_aliases={1: 0},
)


def kernel(x_embed, prompt_pool, prompt_key):
    out0, sim, pnorm, xsum = _stream_call(x_embed, prompt_key)
    prompts, idx128, bkn, rs = _get_sc_call()(
        sim.reshape(B * POOL), pnorm, xsum.reshape(D),
        prompt_pool.reshape(POOL * LEN, D))
    out = _insert_call(prompts, out0)
    return (out, rs[0], sim, idx128.reshape(B, K), bkn)
